# baseline (speedup 1.0000x reference)
import functools

import jax
import jax.numpy as jnp
from jax import lax
from jax.experimental import pallas as pl
from jax.experimental.pallas import tpu as pltpu
from jax.experimental.pallas import tpu_sc as plsc

N = 10000
E = 320000
D = 128
H = 128
C = 16
G = 128
K = 3

NC = 2
NS = 16
NW = NC * NS

CH = 128
CPW = 80
E_PAD = NW * CH * CPW
N_PROP = 10112
SUB_P = N_PROP // NS
N_DEG = 10240
SUB_D = N_DEG // NS
DUMMY_ROW = N

@functools.cache
def _mesh():
    return plsc.VectorSubcoreMesh(core_axis_name="c", subcore_axis_name="s",
                                  num_cores=NC, num_subcores=NS)


@functools.cache
def _cp():
    import dataclasses
    cp = pltpu.CompilerParams()
    if "needs_layout_passes" in pltpu.CompilerParams.__dataclass_fields__:
        cp = dataclasses.replace(cp, needs_layout_passes=False)
    return cp


def _sc_degree(rows_pad):
    @functools.partial(
        pl.kernel,
        out_type=jax.ShapeDtypeStruct((NC, N_DEG), jnp.float32),
        mesh=_mesh(),
        compiler_params=_cp(),
        scratch_types=[
            pltpu.VMEM_SHARED((NS, N_DEG), jnp.float32),
            pltpu.VMEM((CPW * CH,), jnp.int32),
            pltpu.VMEM((N_DEG,), jnp.float32),
            pltpu.VMEM((NS, SUB_D), jnp.float32),
        ],
    )
    def k(rows_hbm, out_hbm, stage, rowv, hist, gath):
        c = lax.axis_index("c")
        s = lax.axis_index("s")
        w = s * NC + c
        pltpu.sync_copy(rows_hbm.at[pl.ds(w * (CPW * CH), CPW * CH)], rowv)

        @pl.loop(0, N_DEG, step=16)
        def _(i):
            hist[pl.ds(i, 16)] = jnp.zeros((16,), jnp.float32)

        ones = jnp.ones((16,), jnp.float32)

        @pl.loop(0, CPW * CH, step=16)
        def _(j):
            plsc.addupdate_scatter(hist, [rowv[pl.ds(j, 16)]], ones)

        pltpu.sync_copy(hist, stage.at[s])
        plsc.subcore_barrier()
        pltpu.sync_copy(stage.at[:, pl.ds(s * SUB_D, SUB_D)], gath)

        @pl.loop(0, SUB_D, step=16)
        def _(i):
            acc16 = gath[0, pl.ds(i, 16)]
            for t in range(1, NS):
                acc16 = acc16 + gath[t, pl.ds(i, 16)]
            hist[pl.ds(i, 16)] = acc16

        pltpu.sync_copy(hist.at[pl.ds(0, SUB_D)],
                        out_hbm.at[c, pl.ds(s * SUB_D, SUB_D)])

    return k(rows_pad)


def _sc_propagate(h, rows3, cols3, zeros_d):
    @functools.partial(
        pl.kernel,
        out_type=jax.ShapeDtypeStruct((NC, N_PROP, D), jnp.float32),
        mesh=_mesh(),
        scratch_types=[
            pltpu.VMEM_SHARED((N_PROP, D), jnp.float32),
            pltpu.VMEM((CH,), jnp.int32),
            pltpu.VMEM((CH,), jnp.int32),
            pltpu.VMEM((CH,), jnp.int32),
            pltpu.VMEM((CH,), jnp.int32),
            pltpu.VMEM((CH, D), jnp.float32),
            pltpu.VMEM((CH, D), jnp.float32),
            pltpu.SemaphoreType.DMA,
            pltpu.SemaphoreType.DMA,
            pltpu.SemaphoreType.DMA,
            pltpu.SemaphoreType.DMA,
            pltpu.SemaphoreType.DMA,
            pltpu.SemaphoreType.DMA,
            pltpu.SemaphoreType.DMA,
            pltpu.SemaphoreType.DMA,
        ],
    )
    def k(h_hbm, rows_hbm, cols_hbm, zeros_hbm, out_hbm,
          acc, colv0, colv1, rowv0, rowv1, b0, b1,
          sg0, sg1, ss0, ss1, sic0, sic1, sir0, sir1):
        c = lax.axis_index("c")
        s = lax.axis_index("s")
        w = s * NC + c
        pltpu.sync_copy(cols_hbm.at[w, 0], colv0)
        pltpu.sync_copy(cols_hbm.at[w, 1], colv1)
        pltpu.sync_copy(rows_hbm.at[w, 0], rowv0)
        pltpu.sync_copy(rows_hbm.at[w, 1], rowv1)
        pltpu.sync_copy(zeros_hbm, acc.at[pl.ds(s * SUB_P, SUB_P)])
        plsc.subcore_barrier()

        def wait_g(sem, idxref, buf):
            pltpu.make_async_copy(h_hbm.at[idxref], buf, sem).wait()

        def wait_s(sem, buf, idxref):
            pltpu.make_async_copy(buf, acc.at[idxref], sem).wait()

        def wait_i(sem, buf):
            pltpu.make_async_copy(cols_hbm.at[w, 0], buf, sem).wait()

        pltpu.async_copy(h_hbm.at[colv0], b0, sg0)
        pltpu.async_copy(h_hbm.at[colv1], b1, sg1)

        @pl.loop(0, CPW, step=2)
        def _(i):
            wait_g(sg0, colv0, b0)

            @pl.when(i + 2 < CPW)
            def _():
                pltpu.async_copy(cols_hbm.at[w, i + 2], colv0, sic0)

            @pl.when(i >= 2)
            def _():
                wait_i(sir0, rowv0)

            pltpu.async_copy(b0, acc.at[rowv0], ss0, add=True)

            wait_g(sg1, colv1, b1)

            @pl.when(i + 3 < CPW)
            def _():
                pltpu.async_copy(cols_hbm.at[w, i + 3], colv1, sic1)

            @pl.when(i >= 2)
            def _():
                wait_i(sir1, rowv1)

            pltpu.async_copy(b1, acc.at[rowv1], ss1, add=True)

            wait_s(ss0, b0, rowv0)

            @pl.when(i + 2 < CPW)
            def _():
                pltpu.async_copy(rows_hbm.at[w, i + 2], rowv0, sir0)
                wait_i(sic0, colv0)
                pltpu.async_copy(h_hbm.at[colv0], b0, sg0)

            wait_s(ss1, b1, rowv1)

            @pl.when(i + 3 < CPW)
            def _():
                pltpu.async_copy(rows_hbm.at[w, i + 3], rowv1, sir1)
                wait_i(sic1, colv1)
                pltpu.async_copy(h_hbm.at[colv1], b1, sg1)

        plsc.subcore_barrier()
        pltpu.sync_copy(acc.at[pl.ds(s * SUB_P, SUB_P)],
                        out_hbm.at[c, pl.ds(s * SUB_P, SUB_P)])

    return k(h, rows3, cols3, zeros_d)


BN = 1000


def _tc_scale(t_parts, h, deg2):
    def body(t0, t1, h_ref, d0, d1, o_ref):
        d = (d0[0, 0, 0, :] + d1[0, 0, 0, :] + 1.0).reshape(BN, 1)
        o_ref[...] = (t0[0] + t1[0] + h_ref[...]) / d

    return pl.pallas_call(
        body,
        grid=(N // BN,),
        in_specs=[
            pl.BlockSpec((1, BN, D), lambda i: (0, i, 0)),
            pl.BlockSpec((1, BN, D), lambda i: (1, i, 0)),
            pl.BlockSpec((BN, D), lambda i: (i, 0)),
            pl.BlockSpec((1, 1, 1, BN), lambda i: (0, i, 0, 0)),
            pl.BlockSpec((1, 1, 1, BN), lambda i: (1, i, 0, 0)),
        ],
        out_specs=pl.BlockSpec((BN, D), lambda i: (i, 0)),
        out_shape=jax.ShapeDtypeStruct((N, D), jnp.float32),
    )(t_parts, t_parts, h, deg2, deg2)


def _tc_mlp_pool(t_parts, h, deg2, batch3, W1, b1, W2, b2, Wc, bc):
    nblk = N // BN

    def body(t0, t1, h_ref, d0, d1, b_ref, W1r, b1r, W2r, b2r, Wcr, bcr,
             o_ref, accr, cntr):
        i = pl.program_id(0)

        @pl.when(i == 0)
        def _():
            accr[...] = jnp.zeros_like(accr)
            cntr[...] = jnp.zeros_like(cntr)

        d = (d0[0, 0, 0, :] + d1[0, 0, 0, :] + 1.0).reshape(BN, 1)
        h3 = (t0[0] + t1[0] + h_ref[...]) / d
        a = jnp.dot(h3, W1r[...], preferred_element_type=jnp.float32) + b1r[...]
        a = jnp.maximum(a, 0.0)
        bvals = b_ref[...].reshape(1, BN)
        onehot_t = (lax.broadcasted_iota(jnp.int32, (G, BN), 0) == bvals
                    ).astype(jnp.float32)
        accr[...] += jnp.dot(onehot_t, a, preferred_element_type=jnp.float32)
        cntr[...] += jnp.sum(onehot_t, axis=1, keepdims=True)

        @pl.when(i == nblk - 1)
        def _():
            pooled = accr[...] / jnp.maximum(cntr[...], 1.0)
            p2 = jnp.dot(pooled, W2r[...], preferred_element_type=jnp.float32) + b2r[...]
            o_ref[...] = jnp.dot(p2, Wcr[...], preferred_element_type=jnp.float32) + bcr[...]

    return pl.pallas_call(
        body,
        grid=(nblk,),
        in_specs=[
            pl.BlockSpec((1, BN, D), lambda i: (0, i, 0)),
            pl.BlockSpec((1, BN, D), lambda i: (1, i, 0)),
            pl.BlockSpec((BN, D), lambda i: (i, 0)),
            pl.BlockSpec((1, 1, 1, BN), lambda i: (0, i, 0, 0)),
            pl.BlockSpec((1, 1, 1, BN), lambda i: (1, i, 0, 0)),
            pl.BlockSpec((1, 1, BN), lambda i: (i, 0, 0)),
            pl.BlockSpec((D, H), lambda i: (0, 0)),
            pl.BlockSpec((1, H), lambda i: (0, 0)),
            pl.BlockSpec((H, H), lambda i: (0, 0)),
            pl.BlockSpec((1, H), lambda i: (0, 0)),
            pl.BlockSpec((H, C), lambda i: (0, 0)),
            pl.BlockSpec((1, C), lambda i: (0, 0)),
        ],
        out_specs=pl.BlockSpec((G, C), lambda i: (0, 0)),
        out_shape=jax.ShapeDtypeStruct((G, C), jnp.float32),
        scratch_shapes=[
            pltpu.VMEM((G, H), jnp.float32),
            pltpu.VMEM((G, 1), jnp.float32),
        ],
    )(t_parts, t_parts, h, deg2, deg2, batch3,
      W1, b1.reshape(1, H), W2, b2.reshape(1, H), Wc, bc.reshape(1, C))


def kernel(x, edge_index, batch, W1, b1, W2, b2, Wc, bc):
    rows = edge_index[0]
    cols = edge_index[1]
    pad = E_PAD - E
    rows_pad = jnp.concatenate([rows, jnp.full((pad,), DUMMY_ROW, jnp.int32)])
    cols_pad = jnp.concatenate([cols, jnp.zeros((pad,), jnp.int32)])
    zeros_d = jnp.zeros((SUB_P, D), jnp.float32)
    batch3 = batch.reshape(N // BN, 1, BN)

    rows3 = rows_pad.reshape(NW, CPW, CH)
    cols3 = cols_pad.reshape(NW, CPW, CH)
    deg_parts = _sc_degree(rows_pad)
    deg2 = deg_parts[:, :N].reshape(NC, N // BN, 1, BN)
    h = x
    t_parts = None
    for step in range(K):
        t_parts = _sc_propagate(h, rows3, cols3, zeros_d)
        if step < K - 1:
            h = _tc_scale(t_parts, h, deg2)
    return _tc_mlp_pool(t_parts, h, deg2, batch3, W1, b1, W2, b2, Wc, bc)


# speedup vs baseline: 2.9584x; 2.9584x over previous
import functools

import jax
import jax.numpy as jnp
from jax import lax
from jax.experimental import pallas as pl
from jax.experimental.pallas import tpu as pltpu
from jax.experimental.pallas import tpu_sc as plsc

N = 10000
E = 320000
D = 128
H = 128
C = 16
G = 128
K = 3

NC = 2
NS = 16
NW = NC * NS

CH = 128
CPW = 80
E_PAD = NW * CH * CPW
N_PROP = 10112
SUB_P = N_PROP // NS
N_DEG = 10240
SUB_D = N_DEG // NS
DUMMY_ROW = N

@functools.cache
def _mesh():
    return plsc.VectorSubcoreMesh(core_axis_name="c", subcore_axis_name="s",
                                  num_cores=NC, num_subcores=NS)


@functools.cache
def _cp():
    import dataclasses
    cp = pltpu.CompilerParams()
    if "needs_layout_passes" in pltpu.CompilerParams.__dataclass_fields__:
        cp = dataclasses.replace(cp, needs_layout_passes=False)
    return cp


def _sc_degree(rows_pad):
    @functools.partial(
        pl.kernel,
        out_type=jax.ShapeDtypeStruct((NC, N_DEG), jnp.float32),
        mesh=_mesh(),
        compiler_params=_cp(),
        scratch_types=[
            pltpu.VMEM_SHARED((NS, N_DEG), jnp.float32),
            pltpu.VMEM((CPW * CH,), jnp.int32),
            pltpu.VMEM((N_DEG,), jnp.float32),
            pltpu.VMEM((NS, SUB_D), jnp.float32),
        ],
    )
    def k(rows_hbm, out_hbm, stage, rowv, hist, gath):
        c = lax.axis_index("c")
        s = lax.axis_index("s")
        w = s * NC + c
        pltpu.sync_copy(rows_hbm.at[pl.ds(w * (CPW * CH), CPW * CH)], rowv)

        @pl.loop(0, N_DEG, step=16)
        def _(i):
            hist[pl.ds(i, 16)] = jnp.zeros((16,), jnp.float32)

        ones = jnp.ones((16,), jnp.float32)

        @pl.loop(0, CPW * CH, step=16)
        def _(j):
            plsc.addupdate_scatter(hist, [rowv[pl.ds(j, 16)]], ones)

        pltpu.sync_copy(hist, stage.at[s])
        plsc.subcore_barrier()
        pltpu.sync_copy(stage.at[:, pl.ds(s * SUB_D, SUB_D)], gath)

        @pl.loop(0, SUB_D, step=16)
        def _(i):
            acc16 = gath[0, pl.ds(i, 16)]
            for t in range(1, NS):
                acc16 = acc16 + gath[t, pl.ds(i, 16)]
            hist[pl.ds(i, 16)] = acc16

        pltpu.sync_copy(hist.at[pl.ds(0, SUB_D)],
                        out_hbm.at[c, pl.ds(s * SUB_D, SUB_D)])

    return k(rows_pad)


def _sc_propagate(h, rows3, cols3, zeros_d):
    @functools.partial(
        pl.kernel,
        out_type=jax.ShapeDtypeStruct((NC, N_PROP, D), jnp.float32),
        mesh=_mesh(),
        scratch_types=[
            pltpu.VMEM_SHARED((N_PROP, D), jnp.float32),
            pltpu.VMEM((CH,), jnp.int32),
            pltpu.VMEM((CH,), jnp.int32),
            pltpu.VMEM((CH,), jnp.int32),
            pltpu.VMEM((CH,), jnp.int32),
            pltpu.VMEM((CH, D), jnp.float32),
            pltpu.VMEM((CH, D), jnp.float32),
            pltpu.SemaphoreType.DMA,
            pltpu.SemaphoreType.DMA,
            pltpu.SemaphoreType.DMA,
            pltpu.SemaphoreType.DMA,
            pltpu.SemaphoreType.DMA,
            pltpu.SemaphoreType.DMA,
            pltpu.SemaphoreType.DMA,
            pltpu.SemaphoreType.DMA,
        ],
    )
    def k(h_hbm, rows_hbm, cols_hbm, zeros_hbm, out_hbm,
          acc, colv0, colv1, rowv0, rowv1, b0, b1,
          sg0, sg1, ss0, ss1, sic0, sic1, sir0, sir1):
        c = lax.axis_index("c")
        s = lax.axis_index("s")
        w = s * NC + c
        pltpu.sync_copy(cols_hbm.at[w, 0], colv0)
        pltpu.sync_copy(cols_hbm.at[w, 1], colv1)
        pltpu.sync_copy(rows_hbm.at[w, 0], rowv0)
        pltpu.sync_copy(rows_hbm.at[w, 1], rowv1)
        pltpu.sync_copy(zeros_hbm, acc.at[pl.ds(s * SUB_P, SUB_P)])
        plsc.subcore_barrier()

        def wait_g(sem, idxref, buf):
            pltpu.make_async_copy(h_hbm.at[idxref], buf, sem).wait()

        def wait_s(sem, buf, idxref):
            pltpu.make_async_copy(buf, acc.at[idxref], sem).wait()

        def wait_i(sem, buf):
            pltpu.make_async_copy(cols_hbm.at[w, 0], buf, sem).wait()

        pltpu.async_copy(h_hbm.at[colv0], b0, sg0)
        pltpu.async_copy(h_hbm.at[colv1], b1, sg1)

        @pl.loop(0, CPW, step=2)
        def _(i):
            wait_g(sg0, colv0, b0)

            @pl.when(i + 2 < CPW)
            def _():
                pltpu.async_copy(cols_hbm.at[w, i + 2], colv0, sic0)

            @pl.when(i >= 2)
            def _():
                wait_i(sir0, rowv0)

            pltpu.async_copy(b0, acc.at[rowv0], ss0, add=True)

            wait_g(sg1, colv1, b1)

            @pl.when(i + 3 < CPW)
            def _():
                pltpu.async_copy(cols_hbm.at[w, i + 3], colv1, sic1)

            @pl.when(i >= 2)
            def _():
                wait_i(sir1, rowv1)

            pltpu.async_copy(b1, acc.at[rowv1], ss1, add=True)

            wait_s(ss0, b0, rowv0)

            @pl.when(i + 2 < CPW)
            def _():
                pltpu.async_copy(rows_hbm.at[w, i + 2], rowv0, sir0)
                wait_i(sic0, colv0)
                pltpu.async_copy(h_hbm.at[colv0], b0, sg0)

            wait_s(ss1, b1, rowv1)

            @pl.when(i + 3 < CPW)
            def _():
                pltpu.async_copy(rows_hbm.at[w, i + 3], rowv1, sir1)
                wait_i(sic1, colv1)
                pltpu.async_copy(h_hbm.at[colv1], b1, sg1)

        plsc.subcore_barrier()
        pltpu.sync_copy(acc.at[pl.ds(s * SUB_P, SUB_P)],
                        out_hbm.at[c, pl.ds(s * SUB_P, SUB_P)])

    return k(h, rows3, cols3, zeros_d)


BN = 1000


def _tc_scale(t_parts, h, deg2):
    def body(t0, t1, h_ref, d0, d1, o_ref):
        d = (d0[0, 0, 0, :] + d1[0, 0, 0, :] + 1.0).reshape(BN, 1)
        o_ref[...] = (t0[0] + t1[0] + h_ref[...]) / d

    return pl.pallas_call(
        body,
        grid=(N // BN,),
        in_specs=[
            pl.BlockSpec((1, BN, D), lambda i: (0, i, 0)),
            pl.BlockSpec((1, BN, D), lambda i: (1, i, 0)),
            pl.BlockSpec((BN, D), lambda i: (i, 0)),
            pl.BlockSpec((1, 1, 1, BN), lambda i: (0, i, 0, 0)),
            pl.BlockSpec((1, 1, 1, BN), lambda i: (1, i, 0, 0)),
        ],
        out_specs=pl.BlockSpec((BN, D), lambda i: (i, 0)),
        out_shape=jax.ShapeDtypeStruct((N, D), jnp.float32),
    )(t_parts, t_parts, h, deg2, deg2)


def _tc_mlp_pool(t_parts, h, deg2, batch3, W1, b1, W2, b2, Wc, bc):
    nblk = N // BN

    def body(t0, t1, h_ref, d0, d1, b_ref, W1r, b1r, W2r, b2r, Wcr, bcr,
             o_ref, accr, cntr):
        i = pl.program_id(0)

        @pl.when(i == 0)
        def _():
            accr[...] = jnp.zeros_like(accr)
            cntr[...] = jnp.zeros_like(cntr)

        d = (d0[0, 0, 0, :] + d1[0, 0, 0, :] + 1.0).reshape(BN, 1)
        h3 = (t0[0] + t1[0] + h_ref[...]) / d
        a = jnp.dot(h3, W1r[...], preferred_element_type=jnp.float32) + b1r[...]
        a = jnp.maximum(a, 0.0)
        bvals = b_ref[...].reshape(1, BN)
        onehot_t = (lax.broadcasted_iota(jnp.int32, (G, BN), 0) == bvals
                    ).astype(jnp.float32)
        accr[...] += jnp.dot(onehot_t, a, preferred_element_type=jnp.float32)
        cntr[...] += jnp.sum(onehot_t, axis=1, keepdims=True)

        @pl.when(i == nblk - 1)
        def _():
            pooled = accr[...] / jnp.maximum(cntr[...], 1.0)
            p2 = jnp.dot(pooled, W2r[...], preferred_element_type=jnp.float32) + b2r[...]
            o_ref[...] = jnp.dot(p2, Wcr[...], preferred_element_type=jnp.float32) + bcr[...]

    return pl.pallas_call(
        body,
        grid=(nblk,),
        in_specs=[
            pl.BlockSpec((1, BN, D), lambda i: (0, i, 0)),
            pl.BlockSpec((1, BN, D), lambda i: (1, i, 0)),
            pl.BlockSpec((BN, D), lambda i: (i, 0)),
            pl.BlockSpec((1, 1, 1, BN), lambda i: (0, i, 0, 0)),
            pl.BlockSpec((1, 1, 1, BN), lambda i: (1, i, 0, 0)),
            pl.BlockSpec((1, 1, BN), lambda i: (i, 0, 0)),
            pl.BlockSpec((D, H), lambda i: (0, 0)),
            pl.BlockSpec((1, H), lambda i: (0, 0)),
            pl.BlockSpec((H, H), lambda i: (0, 0)),
            pl.BlockSpec((1, H), lambda i: (0, 0)),
            pl.BlockSpec((H, C), lambda i: (0, 0)),
            pl.BlockSpec((1, C), lambda i: (0, 0)),
        ],
        out_specs=pl.BlockSpec((G, C), lambda i: (0, 0)),
        out_shape=jax.ShapeDtypeStruct((G, C), jnp.float32),
        scratch_shapes=[
            pltpu.VMEM((G, H), jnp.float32),
            pltpu.VMEM((G, 1), jnp.float32),
        ],
    )(t_parts, t_parts, h, deg2, deg2, batch3,
      W1, b1.reshape(1, H), W2, b2.reshape(1, H), Wc, bc.reshape(1, C))


def kernel(x, edge_index, batch, W1, b1, W2, b2, Wc, bc):
    rows = edge_index[0]
    cols = edge_index[1]
    pad = E_PAD - E
    pad_idx = jnp.arange(pad, dtype=jnp.int32)
    rows_pad = jnp.concatenate([rows, DUMMY_ROW + pad_idx % (N_PROP - N)])
    cols_pad = jnp.concatenate([cols, pad_idx % N])
    zeros_d = jnp.zeros((SUB_P, D), jnp.float32)
    batch3 = batch.reshape(N // BN, 1, BN)

    rows3 = rows_pad.reshape(NW, CPW, CH)
    cols3 = cols_pad.reshape(NW, CPW, CH)
    deg_parts = _sc_degree(rows_pad)
    deg2 = deg_parts[:, :N].reshape(NC, N // BN, 1, BN)
    h = x
    t_parts = None
    for step in range(K):
        t_parts = _sc_propagate(h, rows3, cols3, zeros_d)
        if step < K - 1:
            h = _tc_scale(t_parts, h, deg2)
    return _tc_mlp_pool(t_parts, h, deg2, batch3, W1, b1, W2, b2, Wc, bc)


# speedup vs baseline: 3.9010x; 1.3186x over previous
import functools

import jax
import jax.numpy as jnp
from jax import lax
from jax.experimental import pallas as pl
from jax.experimental.pallas import tpu as pltpu
from jax.experimental.pallas import tpu_sc as plsc

N = 10000
E = 320000
D = 128
H = 128
C = 16
G = 128
K = 3

NC = 2
NS = 16
NW = NC * NS

CH = 64
CPW = 160
E_PAD = NW * CH * CPW
N_PROP = 10112
SUB_P = N_PROP // NS
N_DEG = 10240
SUB_D = N_DEG // NS
DUMMY_ROW = N

@functools.cache
def _mesh():
    return plsc.VectorSubcoreMesh(core_axis_name="c", subcore_axis_name="s",
                                  num_cores=NC, num_subcores=NS)


@functools.cache
def _cp():
    import dataclasses
    cp = pltpu.CompilerParams()
    if "needs_layout_passes" in pltpu.CompilerParams.__dataclass_fields__:
        cp = dataclasses.replace(cp, needs_layout_passes=False)
    return cp


def _sc_degree(rows_pad):
    @functools.partial(
        pl.kernel,
        out_type=jax.ShapeDtypeStruct((NC, N_DEG), jnp.float32),
        mesh=_mesh(),
        compiler_params=_cp(),
        scratch_types=[
            pltpu.VMEM_SHARED((NS, N_DEG), jnp.float32),
            pltpu.VMEM((CPW * CH,), jnp.int32),
            pltpu.VMEM((N_DEG,), jnp.float32),
            pltpu.VMEM((NS, SUB_D), jnp.float32),
        ],
    )
    def k(rows_hbm, out_hbm, stage, rowv, hist, gath):
        c = lax.axis_index("c")
        s = lax.axis_index("s")
        w = s * NC + c
        pltpu.sync_copy(rows_hbm.at[pl.ds(w * (CPW * CH), CPW * CH)], rowv)

        @pl.loop(0, N_DEG, step=16)
        def _(i):
            hist[pl.ds(i, 16)] = jnp.zeros((16,), jnp.float32)

        ones = jnp.ones((16,), jnp.float32)

        @pl.loop(0, CPW * CH, step=16)
        def _(j):
            plsc.addupdate_scatter(hist, [rowv[pl.ds(j, 16)]], ones)

        pltpu.sync_copy(hist, stage.at[s])
        plsc.subcore_barrier()
        pltpu.sync_copy(stage.at[:, pl.ds(s * SUB_D, SUB_D)], gath)

        @pl.loop(0, SUB_D, step=16)
        def _(i):
            acc16 = gath[0, pl.ds(i, 16)]
            for t in range(1, NS):
                acc16 = acc16 + gath[t, pl.ds(i, 16)]
            hist[pl.ds(i, 16)] = acc16

        pltpu.sync_copy(hist.at[pl.ds(0, SUB_D)],
                        out_hbm.at[c, pl.ds(s * SUB_D, SUB_D)])

    return k(rows_pad)


def _sc_propagate(h, rows3, cols3, zeros_d):
    R = 4
    NT = (CPW + 2 + R - 1) // R

    @functools.partial(
        pl.kernel,
        out_type=jax.ShapeDtypeStruct((NC, N_PROP, D), jnp.float32),
        mesh=_mesh(),
        scratch_types=(
            [pltpu.VMEM_SHARED((N_PROP, D), jnp.float32)]
            + [pltpu.VMEM((CH,), jnp.int32) for _ in range(2 * R)]
            + [pltpu.VMEM((CH, D), jnp.float32) for _ in range(R)]
            + [pltpu.SemaphoreType.DMA for _ in range(4 * R)]
        ),
    )
    def k(h_hbm, rows_hbm, cols_hbm, zeros_hbm, out_hbm, acc, *sc):
        colv = sc[0:R]
        rowv = sc[R:2 * R]
        buf = sc[2 * R:3 * R]
        sg = sc[3 * R:4 * R]
        ss = sc[4 * R:5 * R]
        sic = sc[5 * R:6 * R]
        sir = sc[6 * R:7 * R]
        c = lax.axis_index("c")
        s = lax.axis_index("s")
        w = s * NC + c
        for u in range(R):
            pltpu.sync_copy(cols_hbm.at[w, u], colv[u])
        pltpu.sync_copy(zeros_hbm, acc.at[pl.ds(s * SUB_P, SUB_P)])
        plsc.subcore_barrier()

        @pl.loop(0, NT)
        def _(it):
            for u in range(R):
                j = it * R + u
                b = u
                b2 = (u - 2) % R

                @pl.when(j >= R)
                def _():
                    pltpu.make_async_copy(buf[b], acc.at[rowv[b]], ss[b]).wait()

                @pl.when(j < CPW)
                def _():
                    pltpu.async_copy(rows_hbm.at[w, j], rowv[b], sir[b])

                    @pl.when(j >= R)
                    def _():
                        pltpu.make_async_copy(cols_hbm.at[w, 0], colv[b],
                                              sic[b]).wait()

                    pltpu.async_copy(h_hbm.at[colv[b]], buf[b], sg[b])

                @pl.when(jnp.logical_and(j >= 2, j < CPW + 2))
                def _():
                    pltpu.make_async_copy(h_hbm.at[colv[b2]], buf[b2],
                                          sg[b2]).wait()

                    @pl.when(jnp.logical_and(j >= 2, j + 2 < CPW))
                    def _():
                        pltpu.async_copy(cols_hbm.at[w, j + 2], colv[b2],
                                         sic[b2])

                    pltpu.make_async_copy(cols_hbm.at[w, 0], rowv[b2],
                                          sir[b2]).wait()
                    pltpu.async_copy(buf[b2], acc.at[rowv[b2]], ss[b2],
                                     add=True)

        plsc.subcore_barrier()
        pltpu.sync_copy(acc.at[pl.ds(s * SUB_P, SUB_P)],
                        out_hbm.at[c, pl.ds(s * SUB_P, SUB_P)])

    return k(h, rows3, cols3, zeros_d)


BN = 1000


def _tc_scale(t_parts, h, deg2):
    def body(t0, t1, h_ref, d0, d1, o_ref):
        d = (d0[0, 0, 0, :] + d1[0, 0, 0, :] + 1.0).reshape(BN, 1)
        o_ref[...] = (t0[0] + t1[0] + h_ref[...]) / d

    return pl.pallas_call(
        body,
        grid=(N // BN,),
        in_specs=[
            pl.BlockSpec((1, BN, D), lambda i: (0, i, 0)),
            pl.BlockSpec((1, BN, D), lambda i: (1, i, 0)),
            pl.BlockSpec((BN, D), lambda i: (i, 0)),
            pl.BlockSpec((1, 1, 1, BN), lambda i: (0, i, 0, 0)),
            pl.BlockSpec((1, 1, 1, BN), lambda i: (1, i, 0, 0)),
        ],
        out_specs=pl.BlockSpec((BN, D), lambda i: (i, 0)),
        out_shape=jax.ShapeDtypeStruct((N, D), jnp.float32),
    )(t_parts, t_parts, h, deg2, deg2)


def _tc_mlp_pool(t_parts, h, deg2, batch3, W1, b1, W2, b2, Wc, bc):
    nblk = N // BN

    def body(t0, t1, h_ref, d0, d1, b_ref, W1r, b1r, W2r, b2r, Wcr, bcr,
             o_ref, accr, cntr):
        i = pl.program_id(0)

        @pl.when(i == 0)
        def _():
            accr[...] = jnp.zeros_like(accr)
            cntr[...] = jnp.zeros_like(cntr)

        d = (d0[0, 0, 0, :] + d1[0, 0, 0, :] + 1.0).reshape(BN, 1)
        h3 = (t0[0] + t1[0] + h_ref[...]) / d
        a = jnp.dot(h3, W1r[...], preferred_element_type=jnp.float32) + b1r[...]
        a = jnp.maximum(a, 0.0)
        bvals = b_ref[...].reshape(1, BN)
        onehot_t = (lax.broadcasted_iota(jnp.int32, (G, BN), 0) == bvals
                    ).astype(jnp.float32)
        accr[...] += jnp.dot(onehot_t, a, preferred_element_type=jnp.float32)
        cntr[...] += jnp.sum(onehot_t, axis=1, keepdims=True)

        @pl.when(i == nblk - 1)
        def _():
            pooled = accr[...] / jnp.maximum(cntr[...], 1.0)
            p2 = jnp.dot(pooled, W2r[...], preferred_element_type=jnp.float32) + b2r[...]
            o_ref[...] = jnp.dot(p2, Wcr[...], preferred_element_type=jnp.float32) + bcr[...]

    return pl.pallas_call(
        body,
        grid=(nblk,),
        in_specs=[
            pl.BlockSpec((1, BN, D), lambda i: (0, i, 0)),
            pl.BlockSpec((1, BN, D), lambda i: (1, i, 0)),
            pl.BlockSpec((BN, D), lambda i: (i, 0)),
            pl.BlockSpec((1, 1, 1, BN), lambda i: (0, i, 0, 0)),
            pl.BlockSpec((1, 1, 1, BN), lambda i: (1, i, 0, 0)),
            pl.BlockSpec((1, 1, BN), lambda i: (i, 0, 0)),
            pl.BlockSpec((D, H), lambda i: (0, 0)),
            pl.BlockSpec((1, H), lambda i: (0, 0)),
            pl.BlockSpec((H, H), lambda i: (0, 0)),
            pl.BlockSpec((1, H), lambda i: (0, 0)),
            pl.BlockSpec((H, C), lambda i: (0, 0)),
            pl.BlockSpec((1, C), lambda i: (0, 0)),
        ],
        out_specs=pl.BlockSpec((G, C), lambda i: (0, 0)),
        out_shape=jax.ShapeDtypeStruct((G, C), jnp.float32),
        scratch_shapes=[
            pltpu.VMEM((G, H), jnp.float32),
            pltpu.VMEM((G, 1), jnp.float32),
        ],
    )(t_parts, t_parts, h, deg2, deg2, batch3,
      W1, b1.reshape(1, H), W2, b2.reshape(1, H), Wc, bc.reshape(1, C))


def kernel(x, edge_index, batch, W1, b1, W2, b2, Wc, bc):
    rows = edge_index[0]
    cols = edge_index[1]
    pad = E_PAD - E
    pad_idx = jnp.arange(pad, dtype=jnp.int32)
    rows_pad = jnp.concatenate([rows, DUMMY_ROW + pad_idx % (N_PROP - N)])
    cols_pad = jnp.concatenate([cols, pad_idx % N])
    zeros_d = jnp.zeros((SUB_P, D), jnp.float32)
    batch3 = batch.reshape(N // BN, 1, BN)

    rows3 = rows_pad.reshape(NW, CPW, CH)
    cols3 = cols_pad.reshape(NW, CPW, CH)
    deg_parts = _sc_degree(rows_pad)
    deg2 = deg_parts[:, :N].reshape(NC, N // BN, 1, BN)
    h = x
    t_parts = None
    for step in range(K):
        t_parts = _sc_propagate(h, rows3, cols3, zeros_d)
        if step < K - 1:
            h = _tc_scale(t_parts, h, deg2)
    return _tc_mlp_pool(t_parts, h, deg2, batch3, W1, b1, W2, b2, Wc, bc)
